# baseline (speedup 1.0000x reference)
import functools

import jax
import jax.numpy as jnp
from jax import lax
from jax.experimental import pallas as pl
from jax.experimental.pallas import tpu as pltpu
from jax.experimental.pallas import tpu_sc as plsc

VOCAB = 1000000
EMBED_DIM = 32
BATCH = 16384
N_FIELDS = 26

_B = BATCH * N_FIELDS
_LINE = 128
_LINES = VOCAB // 4
_GRP = 208
_C0 = 112
_C1 = _GRP - _C0


def _make_kernel():
    info = plsc.get_sparse_core_info()
    nc, ns = info.num_cores, info.num_subcores
    nw = nc * ns
    b_pw = _B // nw
    n_groups = b_pw // _GRP
    n_gpairs = n_groups // 2

    mesh = plsc.VectorSubcoreMesh(core_axis_name="c", subcore_axis_name="s")

    @functools.partial(
        pl.kernel,
        mesh=mesh,
        out_type=jax.ShapeDtypeStruct((BATCH, N_FIELDS, EMBED_DIM), jnp.float32),
        scratch_types=[
            pltpu.VMEM((b_pw,), jnp.int32),
            pltpu.VMEM((b_pw,), jnp.int32),
            pltpu.VMEM((_C0, _LINE), jnp.float32),
            pltpu.VMEM((_C1, _LINE), jnp.float32),
            pltpu.VMEM((_GRP, EMBED_DIM), jnp.float32),
            pltpu.VMEM((_GRP, EMBED_DIM), jnp.float32),
            pltpu.SemaphoreType.DMA,
            pltpu.SemaphoreType.DMA,
            pltpu.SemaphoreType.DMA,
            pltpu.SemaphoreType.DMA,
        ],
    )
    def emb_kernel(idx_hbm, table_hbm, out_hbm,
                   idx_v, lidx_v, ln0_v, ln1_v, ob0_v, ob1_v,
                   sg0, sg1, sw0, sw1):
        wid = lax.axis_index("s") * nc + lax.axis_index("c")
        b0 = wid * b_pw

        pltpu.sync_copy(idx_hbm.at[pl.ds(b0, b_pw)], idx_v)

        def clamp_body(r, _):
            for c in range(8):
                v = idx_v[pl.ds(r * 128 + c * 16, 16)]
                cv = jnp.maximum(v - 1, 0)
                lidx_v[pl.ds(r * 128 + c * 16, 16)] = (
                    lax.shift_right_logical(cv, 2))
            return ()
        lax.fori_loop(0, b_pw // 128, clamp_body, ())

        lnbufs = (ln0_v, ln1_v)
        gsems = (sg0, sg1)
        wsems = (sw0, sw1)
        obufs = (ob0_v, ob1_v)
        chunk_of = ((0, _C0), (_C0, _C1))

        def gather(g, slot):
            off, n = chunk_of[slot]
            return pltpu.async_copy(
                table_hbm.at[lidx_v.at[pl.ds(g * _GRP + off, n)]],
                lnbufs[slot],
                gsems[slot],
            )

        def gwait(g, slot):
            off, n = chunk_of[slot]
            pltpu.make_async_copy(
                table_hbm.at[lidx_v.at[pl.ds(g * _GRP + off, n)]],
                lnbufs[slot],
                gsems[slot],
            ).wait()

        item0 = wid * (b_pw // N_FIELDS)

        def wb(g, p):
            for i in range(_GRP // N_FIELDS):
                pltpu.async_copy(
                    obufs[p].at[pl.ds(i * N_FIELDS, N_FIELDS)],
                    out_hbm.at[item0 + g * (_GRP // N_FIELDS) + i],
                    wsems[p],
                )

        def wb_wait(g, p):
            for i in range(_GRP // N_FIELDS):
                pltpu.make_async_copy(
                    obufs[p].at[pl.ds(i * N_FIELDS, N_FIELDS)],
                    out_hbm.at[item0 + g * (_GRP // N_FIELDS) + i],
                    wsems[p],
                ).wait()

        def select_chunk(g, slot, p):
            off, n = chunk_of[slot]
            lines = lnbufs[slot]
            obuf = obufs[p]
            for k in range(n // 16):
                pos = g * _GRP + off + k * 16
                v = idx_v[pl.ds(pos, 16)]
                cv = jnp.maximum(v - 1, 0)
                qv = (cv & 3) * EMBED_DIM
                keepf = jnp.minimum(v, 1).astype(jnp.float32)
                for l in range(16):
                    row = k * 16 + l
                    start = qv[l]
                    kf = keepf[l]
                    lo = lines[row, pl.ds(start, 16)] * kf
                    hi = lines[row, pl.ds(start + 16, 16)] * kf
                    obuf[off + row, pl.ds(0, 16)] = lo
                    obuf[off + row, pl.ds(16, 16)] = hi

        gather(0, 0)

        def pair_body(gp, _):
            for p in range(2):
                g = gp * 2 + p

                @pl.when(gp >= 1)
                def _drain(g=g, p=p):
                    wb_wait(g - 2, p)

                gwait(g, 0)
                gather(g, 1)
                select_chunk(g, 0, p)
                gwait(g, 1)

                @pl.when(g + 1 < n_groups)
                def _nxt(g=g):
                    gather(g + 1, 0)

                select_chunk(g, 1, p)
                wb(g, p)
            return ()

        lax.fori_loop(0, n_gpairs, pair_body, ())

        wb_wait(n_groups - 2, 0)
        wb_wait(n_groups - 1, 1)

    return emb_kernel


def kernel(q_idx, embed_para):
    idx_flat = q_idx.astype(jnp.int32).reshape(-1)
    one = idx_flat[0].astype(jnp.float32) * 0.0 + 1.0
    table_lines = embed_para.reshape(_LINES, _LINE) * one
    return _make_kernel()(idx_flat, table_lines)


# speedup vs baseline: 1.2601x; 1.2601x over previous
import functools

import jax
import jax.numpy as jnp
from jax import lax
from jax.experimental import pallas as pl
from jax.experimental.pallas import tpu as pltpu
from jax.experimental.pallas import tpu_sc as plsc

VOCAB = 1000000
EMBED_DIM = 32
BATCH = 16384
N_FIELDS = 26

_B = BATCH * N_FIELDS
_IDX_MINOR = 128
_N_IDX_ROWS = _B // _IDX_MINOR
_G = 8


def _make_kernel():
    info = plsc.get_sparse_core_info()
    nc, ns = info.num_cores, info.num_subcores
    nw = nc * ns
    rows_pw = _N_IDX_ROWS // nw
    n_groups = rows_pw // _G

    mesh = plsc.VectorSubcoreMesh(core_axis_name="c", subcore_axis_name="s")

    @functools.partial(
        pl.kernel,
        mesh=mesh,
        compiler_params=pltpu.CompilerParams(use_tc_tiling_on_sc=False),
        out_type=jax.ShapeDtypeStruct((_B, EMBED_DIM), jnp.float32),
        scratch_types=[
            pltpu.VMEM((rows_pw, _IDX_MINOR), jnp.int32),
            pltpu.VMEM((rows_pw, _IDX_MINOR), jnp.int32),
            pltpu.VMEM((rows_pw + 16,), jnp.int32),
            pltpu.VMEM((_G * _IDX_MINOR, EMBED_DIM), jnp.float32),
            pltpu.SemaphoreType.DMA,
        ],
    )
    def emb_kernel(idx_hbm, flags_hbm, table_hbm, out_hbm,
                   idx_v, cidx_v, flags_v, rows_v, sem):
        wid = lax.axis_index("s") * nc + lax.axis_index("c")
        row0 = wid * rows_pw
        out0 = row0 * _IDX_MINOR

        pltpu.sync_copy(idx_hbm.at[pl.ds(row0, rows_pw)], idx_v)
        flags_v[pl.ds(rows_pw, 16)] = jnp.zeros((16,), jnp.int32)
        pltpu.sync_copy(flags_hbm.at[pl.ds(row0, rows_pw)],
                        flags_v.at[pl.ds(0, rows_pw)])

        zrow = jnp.zeros((16,), jnp.float32)
        lane = lax.iota(jnp.int32, 16)

        def group_body(g, _):
            copies = []
            for j in range(_G):
                r = g * _G + j
                for c in range(_IDX_MINOR // 16):
                    v = idx_v[r, pl.ds(c * 16, 16)]
                    cidx_v[r, pl.ds(c * 16, 16)] = jnp.maximum(v - 1, 0)
                copies.append(pltpu.async_copy(
                    table_hbm.at[cidx_v.at[r]],
                    rows_v.at[pl.ds(j * _IDX_MINOR, _IDX_MINOR)],
                    sem,
                ))
            for cp in copies:
                cp.wait()

            fg = flags_v[pl.ds(g * _G, 16)]
            for j in range(_G):
                r = g * _G + j

                @pl.when(fg[j] != 0)
                def _fix(r=r, j=j):
                    def grp_body(c, _):
                        m = jnp.minimum(idx_v[r, pl.ds(c * 16, 16)], 1)
                        rbase = j * _IDX_MINOR + c * 16
                        for l in range(16):
                            @pl.when(m[l] == 0)
                            def _zero(l=l):
                                rows_v[rbase + l, pl.ds(0, 16)] = zrow
                                rows_v[rbase + l, pl.ds(16, 16)] = zrow
                        return ()
                    lax.fori_loop(0, _IDX_MINOR // 16, grp_body, ())

            pltpu.sync_copy(
                rows_v,
                out_hbm.at[pl.ds(out0 + g * _G * _IDX_MINOR, _G * _IDX_MINOR)],
            )
            return ()

        lax.fori_loop(0, n_groups, group_body, ())

    return emb_kernel


def kernel(q_idx, embed_para):
    idx2d = q_idx.astype(jnp.int32).reshape(_N_IDX_ROWS, _IDX_MINOR)
    flags = (idx2d == 0).any(axis=1).astype(jnp.int32)
    out = _make_kernel()(idx2d, flags, embed_para)
    return out.reshape(BATCH, N_FIELDS, EMBED_DIM)
